# baseline (speedup 1.0000x reference)
import functools

import jax
import jax.numpy as jnp
from jax import lax
from jax.experimental import pallas as pl
from jax.experimental.pallas import tpu as pltpu
from jax.experimental.pallas import tpu_sc as plsc

NC, NS, L = 2, 16, 16
NW = NC * NS

CH = 16
PADP = 24


def _make_sc_kernel(F, PAD, D):
    PER_W = F // NW
    assert PER_W * NW == F and PER_W % 8 == 0
    N_CHUNK = PER_W // CH
    assert N_CHUNK % 2 == 0
    G = 104
    NG = PER_W // G
    assert NG * G == PER_W

    mesh = plsc.VectorSubcoreMesh(core_axis_name="c", subcore_axis_name="s")

    @functools.partial(
        pl.kernel,
        out_type=jax.ShapeDtypeStruct((F, D), jnp.float32),
        mesh=mesh,
        compiler_params=pltpu.CompilerParams(use_tc_tiling_on_sc=False),
        scratch_types=[
            [pltpu.VMEM((G,), jnp.int32) for _ in range(NG)],
            pltpu.VMEM((PER_W, PADP), jnp.int32),
            pltpu.VMEM((PER_W,), jnp.int32),
            pltpu.VMEM((CH * PADP, D), jnp.float32),
            pltpu.VMEM((CH * PADP, D), jnp.float32),
            pltpu.VMEM((PER_W, D), jnp.float32),
            pltpu.SemaphoreType.DMA,
            pltpu.SemaphoreType.DMA,
        ],
    )
    def sc_kernel(ids_hbm, tokens_hbm, lens_hbm, emb_hbm, out_hbm,
                  ids_k, tok_v, lens_v, buf0, buf1, out_v,
                  sem0, sem1):
        wid = lax.axis_index("c") * NS + lax.axis_index("s")
        base = wid * PER_W

        for k in range(NG):
            pltpu.sync_copy(ids_hbm.at[pl.ds(base + k * G, G)], ids_k[k])
        for k in range(NG):
            sl = pl.ds(k * G, G)
            pltpu.sync_copy(tokens_hbm.at[ids_k[k]], tok_v.at[sl])
            pltpu.sync_copy(lens_hbm.at[ids_k[k]], lens_v.at[sl])

        def fire(g, buf, sem):
            for j in range(CH):
                item = g * CH + j
                pltpu.async_copy(
                    emb_hbm.at[tok_v.at[item]],
                    buf.at[pl.ds(j * PADP, PADP)],
                    sem,
                )

        def drain(g, buf, sem):
            for j in range(CH):
                item = g * CH + j
                pltpu.make_async_copy(
                    emb_hbm.at[tok_v.at[item]],
                    buf.at[pl.ds(j * PADP, PADP)],
                    sem,
                ).wait()

        def compute(g, buf):
            invv = 1.0 / lens_v[pl.ds(g * CH, L)].astype(jnp.float32)
            for j in range(CH):
                item = g * CH + j
                acc0 = buf[j * PADP, pl.ds(0, L)]
                acc1 = buf[j * PADP, pl.ds(L, L)]
                for p in range(1, PAD):
                    acc0 = acc0 + buf[j * PADP + p, pl.ds(0, L)]
                    acc1 = acc1 + buf[j * PADP + p, pl.ds(L, L)]
                linv = invv[j]
                out_v[item, pl.ds(0, L)] = acc0 * linv
                out_v[item, pl.ds(L, L)] = acc1 * linv

        fire(0, buf0, sem0)

        def body(t, carry):
            g0 = 2 * t
            g1 = 2 * t + 1
            fire(g1, buf1, sem1)
            drain(g0, buf0, sem0)
            compute(g0, buf0)

            @pl.when(t < N_CHUNK // 2 - 1)
            def _():
                fire(g0 + 2, buf0, sem0)

            drain(g1, buf1, sem1)
            compute(g1, buf1)
            return carry

        lax.fori_loop(0, N_CHUNK // 2, body, 0)

        pltpu.sync_copy(out_v, out_hbm.at[pl.ds(base, PER_W)])

    return sc_kernel


def kernel(ids, tokens, lens, emb_weight):
    BSZ, N = ids.shape
    _, PAD = tokens.shape
    _, D = emb_weight.shape
    F = BSZ * N
    tokens_p = jnp.pad(tokens, ((0, 0), (0, PADP - PAD)))
    sc = _make_sc_kernel(F, PAD, D)
    out = sc(ids.reshape(F), tokens_p, lens, emb_weight)
    return out.reshape(BSZ, N, D)


# speedup vs baseline: 2.4627x; 2.4627x over previous
import functools

import jax
import jax.numpy as jnp
import numpy as np
from jax import lax
from jax.experimental import pallas as pl
from jax.experimental.pallas import tpu as pltpu
from jax.experimental.pallas import tpu_sc as plsc

NC, NS, L = 2, 16, 16
NW = NC * NS

CH = 16
PADP = 32
DESC = 80


def _make_sc_kernel(F, PAD, D):
    PER_W = F // NW
    assert PER_W * NW == F and PER_W % 8 == 0
    N_CHUNK = PER_W // CH
    assert N_CHUNK % 2 == 0
    FLAT = CH * PAD
    assert FLAT % DESC == 0 and FLAT % L == 0
    ND = FLAT // DESC
    G = 104
    NG = PER_W // G
    assert NG * G == PER_W

    mesh = plsc.VectorSubcoreMesh(core_axis_name="c", subcore_axis_name="s")

    @functools.partial(
        pl.kernel,
        out_type=jax.ShapeDtypeStruct((F, D), jnp.float32),
        mesh=mesh,
        compiler_params=pltpu.CompilerParams(
            use_tc_tiling_on_sc=False, needs_layout_passes=False),
        scratch_types=[
            [pltpu.VMEM((G,), jnp.int32) for _ in range(NG)],
            pltpu.VMEM((PER_W, PADP), jnp.int32),
            pltpu.VMEM((PER_W,), jnp.int32),
            pltpu.VMEM((FLAT,), jnp.int32),
            pltpu.VMEM((FLAT,), jnp.int32),
            pltpu.VMEM((FLAT,), jnp.int32),
            pltpu.VMEM((FLAT,), jnp.int32),
            pltpu.VMEM((FLAT, D), jnp.float32),
            pltpu.VMEM((FLAT, D), jnp.float32),
            pltpu.VMEM((PER_W, D), jnp.float32),
            pltpu.SemaphoreType.DMA,
            pltpu.SemaphoreType.DMA,
        ],
    )
    def sc_kernel(ids_hbm, tokens_hbm, lens_hbm, emb_hbm, rc_hbm, out_hbm,
                  ids_k, tok_v, lens_v, rows_v, cols_v, flat0, flat1,
                  buf0, buf1, out_v, sem0, sem1):
        wid = lax.axis_index("c") * NS + lax.axis_index("s")
        base = wid * PER_W

        pltpu.sync_copy(rc_hbm.at[pl.ds(0, FLAT)], rows_v)
        pltpu.sync_copy(rc_hbm.at[pl.ds(FLAT, FLAT)], cols_v)

        for k in range(NG):
            pltpu.sync_copy(ids_hbm.at[pl.ds(base + k * G, G)], ids_k[k])
        for k in range(NG):
            sl = pl.ds(k * G, G)
            pltpu.sync_copy(tokens_hbm.at[ids_k[k]], tok_v.at[sl])
            pltpu.sync_copy(lens_hbm.at[ids_k[k]], lens_v.at[sl])

        def flatten(g, flat):
            row0 = g * CH
            for k in range(FLAT // L):
                sl = pl.ds(k * L, L)
                flat[sl] = plsc.load_gather(
                    tok_v, [rows_v[sl] + row0, cols_v[sl]])

        def fire(flat, buf, sem):
            for s in range(ND):
                pltpu.async_copy(
                    emb_hbm.at[flat.at[pl.ds(s * DESC, DESC)]],
                    buf.at[pl.ds(s * DESC, DESC)],
                    sem,
                )

        def drain(flat, buf, sem):
            for s in range(ND):
                pltpu.make_async_copy(
                    emb_hbm.at[flat.at[pl.ds(s * DESC, DESC)]],
                    buf.at[pl.ds(s * DESC, DESC)],
                    sem,
                ).wait()

        def compute(g, buf):
            invv = 1.0 / lens_v[pl.ds(g * CH, L)].astype(jnp.float32)
            for j in range(CH):
                item = g * CH + j
                acc0 = buf[j * PAD, pl.ds(0, L)]
                acc1 = buf[j * PAD, pl.ds(L, L)]
                for p in range(1, PAD):
                    acc0 = acc0 + buf[j * PAD + p, pl.ds(0, L)]
                    acc1 = acc1 + buf[j * PAD + p, pl.ds(L, L)]
                linv = invv[j]
                out_v[item, pl.ds(0, L)] = acc0 * linv
                out_v[item, pl.ds(L, L)] = acc1 * linv

        flatten(0, flat0)
        fire(flat0, buf0, sem0)

        def body(t, carry):
            g0 = 2 * t
            g1 = 2 * t + 1
            flatten(g1, flat1)
            fire(flat1, buf1, sem1)
            drain(flat0, buf0, sem0)
            compute(g0, buf0)

            @pl.when(t < N_CHUNK // 2 - 1)
            def _():
                flatten(g0 + 2, flat0)
                fire(flat0, buf0, sem0)

            drain(flat1, buf1, sem1)
            compute(g1, buf1)
            return carry

        lax.fori_loop(0, N_CHUNK // 2, body, 0)

        pltpu.sync_copy(out_v, out_hbm.at[pl.ds(base, PER_W)])

    return sc_kernel


def kernel(ids, tokens, lens, emb_weight):
    BSZ, N = ids.shape
    _, PAD = tokens.shape
    _, D = emb_weight.shape
    F = BSZ * N
    FLAT = CH * PAD
    tokens_p = jnp.pad(tokens, ((0, 0), (0, PADP - PAD)))
    q = np.arange(FLAT, dtype=np.int32)
    rc = jnp.asarray(np.concatenate([q // PAD, q % PAD]))
    sc = _make_sc_kernel(F, PAD, D)
    out = sc(ids.reshape(F), tokens_p, lens, emb_weight, rc)
    return out.reshape(BSZ, N, D)
